# baseline (speedup 1.0000x reference)
import jax
import jax.numpy as jnp
from jax.experimental import pallas as pl
from jax.experimental.pallas import tpu as pltpu

N_ROWS = 100000
EMB = 256
HID = 128
OUT = 3
BLK = 4096
NSTEP = pl.cdiv(N_ROWS, BLK)
DEPTH = 3


def _in_copy(src, dst, slot, step, sem):
    start = jnp.minimum(step * BLK, N_ROWS - BLK)
    return pltpu.make_async_copy(
        src.at[pl.ds(start, BLK)], dst.at[slot], sem)


def _out_copy(ob, out_hbm, step, osem):
    start = jnp.minimum(step * BLK, N_ROWS - BLK)
    slot = step % 2 if isinstance(step, int) else jax.lax.rem(step, 2)
    return pltpu.make_async_copy(
        ob.at[slot], out_hbm.at[pl.ds(start, BLK)], osem)


def _body(n_hbm, c_hbm, r_hbm, w_ref, b_ref, wd_ref, bd_ref, out_hbm,
          nb, cb, rb, ob, isem, osem):
    for s in range(DEPTH):
        _in_copy(n_hbm, nb, s, s, isem.at[s, 0]).start()
        _in_copy(c_hbm, cb, s, s, isem.at[s, 1]).start()
        _in_copy(r_hbm, rb, s, s, isem.at[s, 2]).start()

    def step_fn(i, carry):
        slot = jax.lax.rem(i, DEPTH)
        oslot = jax.lax.rem(i, 2)
        _in_copy(n_hbm, nb, slot, i, isem.at[slot, 0]).wait()
        _in_copy(c_hbm, cb, slot, i, isem.at[slot, 1]).wait()
        _in_copy(r_hbm, rb, slot, i, isem.at[slot, 2]).wait()

        dotf = lambda a, b: jax.lax.dot_general(
            a, b, (((1,), (0,)), ((), ())),
            preferred_element_type=jnp.float32)
        bc = dotf(cb[slot], w_ref[:, 2 * HID:])
        nc = dotf(nb[slot], w_ref[:, :HID])
        nc = jnp.maximum(nc + bc[:, :HID] + b_ref[:, :HID], 0.0)
        rc = dotf(rb[slot], w_ref[:, HID:2 * HID])
        rc = jnp.maximum(rc + bc[:, HID:] + b_ref[:, HID:], 0.0)
        res = dotf(nc, wd_ref[:HID]) + dotf(rc, wd_ref[HID:]) + bd_ref[...]

        @pl.when(i >= 2)
        def _():
            _out_copy(ob, out_hbm, i - 2, osem.at[oslot]).wait()

        ob[oslot] = res
        _out_copy(ob, out_hbm, i, osem.at[oslot]).start()

        @pl.when(i < NSTEP - DEPTH)
        def _():
            _in_copy(n_hbm, nb, slot, i + DEPTH, isem.at[slot, 0]).start()
            _in_copy(c_hbm, cb, slot, i + DEPTH, isem.at[slot, 1]).start()
            _in_copy(r_hbm, rb, slot, i + DEPTH, isem.at[slot, 2]).start()
        return carry

    jax.lax.fori_loop(0, NSTEP, step_fn, 0)
    for step in (NSTEP - 2, NSTEP - 1):
        _out_copy(ob, out_hbm, step, osem.at[step % 2]).wait()


def kernel(neighbor, current, remote, W_n, b_n, W_r, b_r, W_d, b_d):
    W_cat = jnp.concatenate(
        [W_n[:EMB], W_r[:EMB], W_n[EMB:], W_r[EMB:]],
        axis=1).astype(jnp.bfloat16)
    b_cat = jnp.concatenate([b_n, b_r]).reshape(1, 2 * HID)
    any_spec = pl.BlockSpec(memory_space=pltpu.MemorySpace.HBM)
    vmem_spec = pl.BlockSpec(memory_space=pltpu.MemorySpace.VMEM)
    out = pl.pallas_call(
        _body,
        in_specs=[any_spec, any_spec, any_spec,
                  vmem_spec, vmem_spec, vmem_spec, vmem_spec],
        out_specs=any_spec,
        out_shape=jax.ShapeDtypeStruct((N_ROWS, OUT), jnp.float32),
        scratch_shapes=[
            pltpu.VMEM((DEPTH, BLK, EMB), jnp.float32),
            pltpu.VMEM((DEPTH, BLK, EMB), jnp.float32),
            pltpu.VMEM((DEPTH, BLK, EMB), jnp.float32),
            pltpu.VMEM((2, BLK, OUT), jnp.float32),
            pltpu.SemaphoreType.DMA((DEPTH, 3)),
            pltpu.SemaphoreType.DMA((2,)),
        ],
        compiler_params=pltpu.CompilerParams(
            vmem_limit_bytes=100 * 1024 * 1024),
    )(neighbor, current, remote, W_cat, b_cat,
      W_d.astype(jnp.bfloat16), b_d.reshape(1, OUT))
    return out


# speedup vs baseline: 1.0053x; 1.0053x over previous
import jax
import jax.numpy as jnp
from jax.experimental import pallas as pl
from jax.experimental.pallas import tpu as pltpu

N_ROWS = 100000
EMB = 256
HID = 128
OUT = 3
BLK = 7168


def _body(n_ref, c_ref, r_ref, w_ref, b_ref, wd_ref, bd_ref, out_ref):
    dotf = lambda a, b: jax.lax.dot_general(
        a, b, (((1,), (0,)), ((), ())),
        preferred_element_type=jnp.float32)
    bc = dotf(c_ref[...], w_ref[:, 2 * HID:])
    nc = dotf(n_ref[...], w_ref[:, :HID])
    nc = jnp.maximum(nc + bc[:, :HID] + b_ref[:, :HID], 0.0)
    rc = dotf(r_ref[...], w_ref[:, HID:2 * HID])
    rc = jnp.maximum(rc + bc[:, HID:] + b_ref[:, HID:], 0.0)
    out = jnp.dot(nc.astype(jnp.bfloat16), wd_ref[:HID],
                  preferred_element_type=jnp.float32)
    out += jnp.dot(rc.astype(jnp.bfloat16), wd_ref[HID:],
                   preferred_element_type=jnp.float32)
    out_ref[...] = out + bd_ref[...]


def kernel(neighbor, current, remote, W_n, b_n, W_r, b_r, W_d, b_d):
    grid = (pl.cdiv(N_ROWS, BLK),)
    row_spec = pl.BlockSpec((BLK, EMB), lambda i: (i, 0))
    full = lambda shape: pl.BlockSpec(shape, lambda i: (0, 0))
    W_cat = jnp.concatenate(
        [W_n[:EMB], W_r[:EMB], W_n[EMB:], W_r[EMB:]],
        axis=1).astype(jnp.bfloat16)
    b_cat = jnp.concatenate([b_n, b_r]).reshape(1, 2 * HID)
    out = pl.pallas_call(
        _body,
        grid=grid,
        in_specs=[
            row_spec, row_spec, row_spec,
            full((EMB, 4 * HID)), full((1, 2 * HID)),
            full((2 * HID, OUT)), full((1, OUT)),
        ],
        out_specs=pl.BlockSpec((BLK, OUT), lambda i: (i, 0)),
        out_shape=jax.ShapeDtypeStruct((N_ROWS, OUT), jnp.float32),
        compiler_params=pltpu.CompilerParams(
            dimension_semantics=(pltpu.ARBITRARY,),
            vmem_limit_bytes=100 * 1024 * 1024),
    )(neighbor, current, remote, W_cat, b_cat,
      W_d.astype(jnp.bfloat16), b_d.reshape(1, OUT))
    return out
